# baseline (speedup 1.0000x reference)
import functools

import jax
import jax.numpy as jnp
from jax import lax
from jax.experimental import pallas as pl
from jax.experimental.pallas import tpu as pltpu
from jax.experimental.pallas import tpu_sc as plsc

_NUM_SC_CORES = 2
_NUM_SC_SUBCORES = 16


def _mlp_body(idx_ref, tab_hbm, w1_hbm, b1_hbm, w2_hbm, b2_hbm, out_ref,
              tab_v, w1_v, b1_v, w2_v, b2_v,
              s_tab, s_w1, s_b1, s_w2, s_b2):
    cps = [
        pltpu.make_async_copy(tab_hbm, tab_v, s_tab),
        pltpu.make_async_copy(w1_hbm, w1_v, s_w1),
        pltpu.make_async_copy(b1_hbm, b1_v, s_b1),
        pltpu.make_async_copy(w2_hbm, w2_v, s_w2),
        pltpu.make_async_copy(b2_hbm, b2_v, s_b2),
    ]
    for cp in cps:
        cp.start()

    idx_row = idx_ref[:, :]
    n_rows = tab_v.shape[0]
    rows = lax.broadcasted_iota(jnp.int32, (n_rows, idx_row.shape[1]), 0)
    onehot_t = (rows == idx_row).astype(jnp.float32)

    cps[0].wait()
    prompt = lax.dot_general(
        onehot_t, tab_v[:, :], (((0,), (0,)), ((), ())),
        preferred_element_type=jnp.float32)

    cps[1].wait()
    cps[2].wait()
    h = jnp.tanh(
        jnp.dot(prompt, w1_v[:, :], preferred_element_type=jnp.float32)
        + b1_v[:, :]
    )

    cps[3].wait()
    cps[4].wait()
    out_ref[:, :] = (
        jnp.dot(h, w2_v[:, :], preferred_element_type=jnp.float32)
        + b2_v[:, :]
    )


def _mlp_result(pre_prompt, embd_table, W1, b1, W2, b2):
    P = pre_prompt.shape[0]
    D, H = W1.shape
    N = embd_table.shape[0]
    hbm = pl.BlockSpec(memory_space=pltpu.MemorySpace.HBM)
    return pl.pallas_call(
        _mlp_body,
        in_specs=[pl.BlockSpec((1, P), lambda: (0, 0)),
                  hbm, hbm, hbm, hbm, hbm],
        out_shape=jax.ShapeDtypeStruct((P, D), jnp.float32),
        scratch_shapes=[
            pltpu.VMEM((N, D), jnp.float32),
            pltpu.VMEM((D, H), jnp.float32),
            pltpu.VMEM((1, H), jnp.float32),
            pltpu.VMEM((H, D), jnp.float32),
            pltpu.VMEM((1, D), jnp.float32),
            pltpu.SemaphoreType.DMA,
            pltpu.SemaphoreType.DMA,
            pltpu.SemaphoreType.DMA,
            pltpu.SemaphoreType.DMA,
            pltpu.SemaphoreType.DMA,
        ],
    )(
        pre_prompt.reshape(1, P),
        embd_table,
        W1,
        b1.reshape(1, H),
        W2,
        b2.reshape(1, D),
    )


def _sc_broadcast(B, P, D):
    mesh = plsc.VectorSubcoreMesh(
        core_axis_name="c", subcore_axis_name="s")

    @functools.partial(
        pl.kernel, mesh=mesh,
        out_type=jax.ShapeDtypeStruct((B, P, D), jnp.float32),
        scratch_types=[
            pltpu.VMEM((D,), jnp.float32),
            pltpu.SemaphoreType.DMA,
        ],
    )
    def k(res_hbm, out_hbm, row_v, sem):
        wid = lax.axis_index("s") * _NUM_SC_CORES + lax.axis_index("c")

        @pl.when(wid < P)
        def _():
            pltpu.sync_copy(res_hbm.at[wid], row_v)
            cps = [pltpu.make_async_copy(row_v, out_hbm.at[b, wid], sem)
                   for b in range(B)]
            for cp in cps:
                cp.start()
            for cp in cps:
                cp.wait()

    return k


def kernel(tokens, batch_size, pre_prompt, embd_table, W1, b1, W2, b2):
    B = tokens.shape[0]
    P = pre_prompt.shape[0]
    D = embd_table.shape[1]
    res = _mlp_result(pre_prompt, embd_table, W1, b1, W2, b2)
    return _sc_broadcast(B, P, D)(res)


# speedup vs baseline: 4.4282x; 4.4282x over previous
import jax
import jax.numpy as jnp
from jax import lax
from jax.experimental import pallas as pl
from jax.experimental.pallas import tpu as pltpu


def _body(idx_ref, tab_hbm, w1_hbm, b1_hbm, w2_hbm, b2_hbm, out_ref,
          tab_v, w1a_v, w1b_v, b1_v, w2a_v, w2b_v, b2_v,
          s_tab, s_w1a, s_w1b, s_b1, s_w2a, s_w2b, s_b2):
    D = w1_hbm.shape[0]
    H = w2_hbm.shape[0]
    dh = D // 2
    hh = H // 2
    cps = [
        pltpu.make_async_copy(tab_hbm, tab_v, s_tab),
        pltpu.make_async_copy(w1_hbm.at[pl.ds(0, dh), :], w1a_v, s_w1a),
        pltpu.make_async_copy(w1_hbm.at[pl.ds(dh, dh), :], w1b_v, s_w1b),
        pltpu.make_async_copy(b1_hbm, b1_v, s_b1),
        pltpu.make_async_copy(w2_hbm.at[pl.ds(0, hh), :], w2a_v, s_w2a),
        pltpu.make_async_copy(w2_hbm.at[pl.ds(hh, hh), :], w2b_v, s_w2b),
        pltpu.make_async_copy(b2_hbm, b2_v, s_b2),
    ]
    for cp in cps:
        cp.start()

    idx_row = idx_ref[:, :]
    n_rows = tab_v.shape[0]
    rows = lax.broadcasted_iota(jnp.int32, (n_rows, idx_row.shape[1]), 0)
    onehot_t = (rows == idx_row).astype(jnp.float32)

    cps[0].wait()
    prompt = lax.dot_general(
        onehot_t, tab_v[:, :], (((0,), (0,)), ((), ())),
        preferred_element_type=jnp.float32)

    cps[1].wait()
    hpre = jnp.dot(prompt[:, 0:dh], w1a_v[:, :],
                   preferred_element_type=jnp.float32)
    cps[2].wait()
    cps[3].wait()
    hpre = hpre + jnp.dot(prompt[:, dh:D], w1b_v[:, :],
                          preferred_element_type=jnp.float32)
    h = jnp.tanh(hpre + b1_v[:, :])

    cps[4].wait()
    oacc = jnp.dot(h[:, 0:hh], w2a_v[:, :],
                   preferred_element_type=jnp.float32)
    cps[5].wait()
    cps[6].wait()
    out_ref[:, :] = (
        oacc
        + jnp.dot(h[:, hh:H], w2b_v[:, :], preferred_element_type=jnp.float32)
        + b2_v[:, :]
    )


def kernel(tokens, batch_size, pre_prompt, embd_table, W1, b1, W2, b2):
    B = tokens.shape[0]
    P = pre_prompt.shape[0]
    D, H = W1.shape
    N = embd_table.shape[0]
    hbm = pl.BlockSpec(memory_space=pltpu.MemorySpace.HBM)
    res = pl.pallas_call(
        _body,
        in_specs=[pl.BlockSpec((1, P), lambda: (0, 0)),
                  hbm, hbm, hbm, hbm, hbm],
        out_shape=jax.ShapeDtypeStruct((P, D), jnp.float32),
        scratch_shapes=[
            pltpu.VMEM((N, D), jnp.float32),
            pltpu.VMEM((D // 2, H), jnp.float32),
            pltpu.VMEM((D // 2, H), jnp.float32),
            pltpu.VMEM((1, H), jnp.float32),
            pltpu.VMEM((H // 2, D), jnp.float32),
            pltpu.VMEM((H // 2, D), jnp.float32),
            pltpu.VMEM((1, D), jnp.float32),
            pltpu.SemaphoreType.DMA,
            pltpu.SemaphoreType.DMA,
            pltpu.SemaphoreType.DMA,
            pltpu.SemaphoreType.DMA,
            pltpu.SemaphoreType.DMA,
            pltpu.SemaphoreType.DMA,
            pltpu.SemaphoreType.DMA,
        ],
    )(
        pre_prompt.reshape(1, P),
        embd_table,
        W1,
        b1.reshape(1, H),
        W2,
        b2.reshape(1, D),
    )
    return jnp.broadcast_to(res[None], (B, P, D))
